# Initial kernel scaffold; baseline (speedup 1.0000x reference)
import jax
import jax.numpy as jnp
from jax.experimental import pallas as pl


def kernel(x, edge_index, params):
    raise NotImplementedError("write your pallas kernel here")


# speedup vs baseline: 1.0544x; 1.0544x over previous
import functools

import jax
import jax.numpy as jnp
from jax import lax
from jax.experimental import pallas as pl
from jax.experimental.pallas import tpu as pltpu
from jax.experimental.pallas import tpu_sc as plsc

N = 50000
N_PAD = 50176
BN = 512
NB = N_PAD // BN
CB = 32
CHUNK = 128
SUPER = 16
E_RAW = 850000
EP = ((E_RAW + 16 * CHUNK - 1) // (16 * CHUNK)) * (16 * CHUNK)
ROWS = EP // CHUNK
TILE_ROWS = ROWS // 16
WID_ROWS = ROWS // 32
NSUP_TILE = TILE_ROWS // SUPER
NSUP_WID = WID_ROWS // SUPER
PT = N_PAD // 16
NZ = 32
ZR = PT // NZ
NBUF = 4
BN_EPS = 1e-5


def _sc_deg_kernel():
    mesh = plsc.VectorSubcoreMesh(core_axis_name="c", subcore_axis_name="s")
    scratch = [
        pltpu.VMEM_SHARED((N_PAD, CB), jnp.float32),
        pltpu.VMEM((ZR, CB), jnp.float32),
        pltpu.VMEM((SUPER, CHUNK), jnp.int32),
        pltpu.VMEM((1, CHUNK, CB), jnp.float32),
    ]

    def body(dstm, out, acc, zbuf, didx, obuf):
        c = lax.axis_index("c")
        s = lax.axis_index("s")

        def zrow(i, _):
            zv = jnp.zeros((16,), jnp.float32)
            zbuf[i, pl.ds(0, 16)] = zv
            zbuf[i, pl.ds(16, 16)] = zv
            return 0
        lax.fori_loop(0, ZR, zrow, 0)

        def orow(i, _):
            ov = jnp.full((16,), 1.0, jnp.float32)
            obuf[0, i, pl.ds(0, 16)] = ov
            obuf[0, i, pl.ds(16, 16)] = ov
            return 0
        lax.fori_loop(0, CHUNK, orow, 0)

        for z in range(NZ):
            pltpu.sync_copy(zbuf, acc.at[pl.ds(s * PT + z * ZR, ZR), :])
        plsc.subcore_barrier()

        rows0 = (s * 2 + c) * WID_ROWS

        def super_body(u, _):
            r0 = rows0 + u * SUPER
            pltpu.sync_copy(dstm.at[pl.ds(r0, SUPER), :], didx)
            for j in range(SUPER):
                pltpu.sync_copy(obuf.at[0], acc.at[didx.at[j]], add=True)
            return 0
        lax.fori_loop(0, NSUP_WID, super_body, 0)
        plsc.subcore_barrier()

        for z in range(NZ):
            r = s * PT + z * ZR
            pltpu.sync_copy(acc.at[pl.ds(r, ZR), :],
                            out.at[c, pl.ds(r, ZR), :])
        plsc.subcore_barrier()

    return pl.kernel(
        body, out_type=jax.ShapeDtypeStruct((2, N_PAD, CB), jnp.float32),
        mesh=mesh, scratch_types=scratch,
        compiler_params=pltpu.CompilerParams(use_tc_tiling_on_sc=False))


def _sc_norm_kernel():
    mesh = plsc.VectorSubcoreMesh(core_axis_name="c", subcore_axis_name="s")
    scratch = [
        pltpu.VMEM((1, CHUNK), jnp.int32),
        pltpu.VMEM((1, CHUNK), jnp.int32),
        pltpu.VMEM((CHUNK, CB), jnp.float32),
        pltpu.VMEM((CHUNK, CB), jnp.float32),
        pltpu.SemaphoreType.DMA,
        pltpu.SemaphoreType.DMA,
    ]

    def body(dinv_t, srcm, dstm, out, sidx, didx, gs, gd, sem1, sem2):
        c = lax.axis_index("c")
        s = lax.axis_index("s")
        rows0 = (s * 2 + c) * WID_ROWS

        def chunk_body(u, _):
            r = rows0 + u
            pltpu.sync_copy(srcm.at[pl.ds(r, 1), :], sidx)
            pltpu.sync_copy(dstm.at[pl.ds(r, 1), :], didx)
            h1 = pltpu.async_copy(dinv_t.at[sidx.at[0]], gs, sem1)
            h2 = pltpu.async_copy(dinv_t.at[didx.at[0]], gd, sem2)
            h1.wait()
            h2.wait()

            def mrow(i, _):
                a0 = gs[i, pl.ds(0, 16)]
                b0 = gd[i, pl.ds(0, 16)]
                gs[i, pl.ds(0, 16)] = a0 * b0
                a1 = gs[i, pl.ds(16, 16)]
                b1 = gd[i, pl.ds(16, 16)]
                gs[i, pl.ds(16, 16)] = a1 * b1
                return 0
            lax.fori_loop(0, CHUNK, mrow, 0)
            pltpu.sync_copy(gs, out.at[pl.ds(r * CHUNK, CHUNK), :])
            return 0
        lax.fori_loop(0, WID_ROWS, chunk_body, 0)

    return pl.kernel(
        body, out_type=jax.ShapeDtypeStruct((EP, CB), jnp.float32),
        mesh=mesh, scratch_types=scratch,
        compiler_params=pltpu.CompilerParams(use_tc_tiling_on_sc=False))


def _sc_gather_kernel(nblk):
    mesh = plsc.VectorSubcoreMesh(core_axis_name="c", subcore_axis_name="s")
    scratch = [
        pltpu.VMEM((SUPER, CHUNK), jnp.int32),
        pltpu.VMEM((NBUF, CHUNK, CB), jnp.float32),
    ] + [pltpu.SemaphoreType.DMA] * NBUF

    def body(table, srcm, out, sidx, gbuf, *sems):
        c = lax.axis_index("c")
        s = lax.axis_index("s")
        rows0 = s * TILE_ROWS

        for bi in range(nblk // 2):
            blk = bi * 2 + c

            def super_body(u, _):
                r0 = rows0 + u * SUPER
                pltpu.sync_copy(srcm.at[pl.ds(r0, SUPER), :], sidx)
                handles = {}
                for j in range(NBUF):
                    handles[j] = pltpu.async_copy(
                        table.at[blk].at[sidx.at[j]], gbuf.at[j], sems[j])
                for j in range(SUPER):
                    handles[j].wait()
                    pltpu.sync_copy(
                        gbuf.at[j % NBUF],
                        out.at[blk, pl.ds((r0 + j) * CHUNK, CHUNK), :])
                    nx = j + NBUF
                    if nx < SUPER:
                        handles[nx] = pltpu.async_copy(
                            table.at[blk].at[sidx.at[nx]],
                            gbuf.at[nx % NBUF], sems[nx % NBUF])
                return 0
            lax.fori_loop(0, NSUP_TILE, super_body, 0)

    return pl.kernel(
        body, out_type=jax.ShapeDtypeStruct((nblk, EP, CB), jnp.float32),
        mesh=mesh, scratch_types=scratch,
        compiler_params=pltpu.CompilerParams(use_tc_tiling_on_sc=False))


def _sc_scatter_kernel(nblk):
    mesh = plsc.VectorSubcoreMesh(core_axis_name="c", subcore_axis_name="s")
    scratch = [
        pltpu.VMEM_SHARED((N_PAD, CB), jnp.float32),
        pltpu.VMEM((ZR, CB), jnp.float32),
        pltpu.VMEM((SUPER, CHUNK), jnp.int32),
        pltpu.VMEM((NBUF, CHUNK, CB), jnp.float32),
    ] + [pltpu.SemaphoreType.DMA] * NBUF

    def body(msg, dstm, out, acc, zbuf, didx, lbuf, *sems):
        c = lax.axis_index("c")
        s = lax.axis_index("s")

        def zrow(i, _):
            zv = jnp.zeros((16,), jnp.float32)
            zbuf[i, pl.ds(0, 16)] = zv
            zbuf[i, pl.ds(16, 16)] = zv
            return 0
        lax.fori_loop(0, ZR, zrow, 0)

        rows0 = s * TILE_ROWS

        for bi in range(nblk // 2):
            blk = bi * 2 + c

            for z in range(NZ):
                pltpu.sync_copy(zbuf, acc.at[pl.ds(s * PT + z * ZR, ZR), :])
            plsc.subcore_barrier()

            def super_body(u, _):
                r0 = rows0 + u * SUPER
                pltpu.sync_copy(dstm.at[pl.ds(r0, SUPER), :], didx)
                handles = {}
                for j in range(NBUF):
                    handles[j] = pltpu.async_copy(
                        msg.at[blk, pl.ds((r0 + j) * CHUNK, CHUNK), :],
                        lbuf.at[j], sems[j])
                for j in range(SUPER):
                    handles[j].wait()
                    pltpu.sync_copy(lbuf.at[j % NBUF],
                                    acc.at[didx.at[j]], add=True)
                    nx = j + NBUF
                    if nx < SUPER:
                        handles[nx] = pltpu.async_copy(
                            msg.at[blk, pl.ds((r0 + nx) * CHUNK, CHUNK), :],
                            lbuf.at[nx % NBUF], sems[nx % NBUF])
                return 0
            lax.fori_loop(0, NSUP_TILE, super_body, 0)
            plsc.subcore_barrier()

            for z in range(NZ):
                r = s * PT + z * ZR
                pltpu.sync_copy(acc.at[pl.ds(r, ZR), :],
                                out.at[blk, pl.ds(r, ZR), :])
            plsc.subcore_barrier()

    return pl.kernel(
        body, out_type=jax.ShapeDtypeStruct((nblk, N_PAD, CB), jnp.float32),
        mesh=mesh, scratch_types=scratch,
        compiler_params=pltpu.CompilerParams(use_tc_tiling_on_sc=False))


@functools.cache
def _sc_deg():
    return _sc_deg_kernel()


@functools.cache
def _sc_norm():
    return _sc_norm_kernel()


@functools.cache
def _sc_gather(nblk):
    return _sc_gather_kernel(nblk)


@functools.cache
def _sc_scatter(nblk):
    return _sc_scatter_kernel(nblk)


def _iota_rows():
    return lax.broadcasted_iota(jnp.int32, (BN, CB), 0)


@functools.cache
def _k_dinv():
    def body(deg_ref, dv_ref):
        i = pl.program_id(0)
        d = deg_ref[0] + deg_ref[1]
        dv = lax.rsqrt(jnp.maximum(d, 1.0))
        row = i * BN + _iota_rows()
        dv_ref[...] = jnp.where(row < N, dv, 0.0)
    return pl.pallas_call(
        body,
        grid=(NB,),
        in_specs=[pl.BlockSpec((2, BN, CB), lambda i: (0, i, 0))],
        out_specs=pl.BlockSpec((BN, CB), lambda i: (i, 0)),
        out_shape=jax.ShapeDtypeStruct((N_PAD, CB), jnp.float32),
    )


@functools.cache
def _k_premm(cin, cout):
    nblk = cout // CB

    def body(x_ref, w_ref, b_ref, hp_ref):
        y = jnp.dot(x_ref[...], w_ref[...],
                    preferred_element_type=jnp.float32)
        hv = y + b_ref[...]
        for k in range(nblk):
            hp_ref[k] = hv[:, k * CB:(k + 1) * CB]
    return pl.pallas_call(
        body,
        grid=(NB,),
        in_specs=[pl.BlockSpec((BN, cin), lambda i: (i, 0)),
                  pl.BlockSpec((cin, cout), lambda i: (0, 0)),
                  pl.BlockSpec((1, cout), lambda i: (0, 0))],
        out_specs=pl.BlockSpec((nblk, BN, CB), lambda i: (0, i, 0)),
        out_shape=jax.ShapeDtypeStruct((nblk, N_PAD, CB), jnp.float32),
    )


@functools.cache
def _k_stats(cout):
    nblk_in = cout // CB

    def body(agg_ref, y_ref, ps_ref, acc):
        i = pl.program_id(0)
        y = jnp.concatenate([agg_ref[k] for k in range(nblk_in)], axis=1)
        y_ref[...] = y

        @pl.when(i == 0)
        def _():
            acc[...] = jnp.zeros((4, 128, cout), jnp.float32)
        for k in range(4):
            acc[k] = acc[k] + y[k * 128:(k + 1) * 128, :]

        @pl.when(i == NB - 1)
        def _():
            a = acc[0]
            for k in range(1, 4):
                a = a + acc[k]
            for st in [1, 2, 4, 8, 16, 32, 64]:
                a = a + pltpu.roll(a, 128 - st, 0)
            ps_ref[...] = a[0].reshape(1, 1, cout)

    return pl.pallas_call(
        body,
        grid=(NB,),
        in_specs=[pl.BlockSpec((nblk_in, BN, CB), lambda i: (0, i, 0))],
        out_specs=[pl.BlockSpec((BN, cout), lambda i: (i, 0)),
                   pl.BlockSpec((1, 1, cout), lambda i: (0, 0, 0))],
        out_shape=[jax.ShapeDtypeStruct((N_PAD, cout), jnp.float32),
                   jax.ShapeDtypeStruct((1, 1, cout), jnp.float32)],
        scratch_shapes=[pltpu.VMEM((4, 128, cout), jnp.float32)],
    )


@functools.cache
def _k_stats2(cout):
    def body(y_ref, ps_ref, pq_ref, acc):
        i = pl.program_id(0)
        m = ps_ref[...].reshape(cout) * (1.0 / N)
        c = y_ref[...] - m[None, :]
        row = i * BN + lax.broadcasted_iota(jnp.int32, (BN, cout), 0)
        sq = jnp.where(row < N, c * c, 0.0)

        @pl.when(i == 0)
        def _():
            acc[...] = jnp.zeros((4, 128, cout), jnp.float32)
        for k in range(4):
            acc[k] = acc[k] + sq[k * 128:(k + 1) * 128, :]

        @pl.when(i == NB - 1)
        def _():
            a = acc[0]
            for k in range(1, 4):
                a = a + acc[k]
            for st in [1, 2, 4, 8, 16, 32, 64]:
                a = a + pltpu.roll(a, 128 - st, 0)
            pq_ref[...] = a[0].reshape(1, 1, cout)

    return pl.pallas_call(
        body,
        grid=(NB,),
        in_specs=[pl.BlockSpec((BN, cout), lambda i: (i, 0)),
                  pl.BlockSpec((1, 1, cout), lambda i: (0, 0, 0))],
        out_specs=pl.BlockSpec((1, 1, cout), lambda i: (0, 0, 0)),
        out_shape=jax.ShapeDtypeStruct((1, 1, cout), jnp.float32),
        scratch_shapes=[pltpu.VMEM((4, 128, cout), jnp.float32)],
    )


@functools.cache
def _k_msgmul(nblk):
    EB = 512

    def body(g_ref, n_ref, o_ref):
        nv = n_ref[...][:, 0:1]
        for k in range(nblk):
            o_ref[k] = g_ref[k] * nv

    return pl.pallas_call(
        body,
        grid=(EP // EB,),
        in_specs=[pl.BlockSpec((nblk, EB, CB), lambda i: (0, i, 0)),
                  pl.BlockSpec((EB, CB), lambda i: (i, 0))],
        out_specs=pl.BlockSpec((nblk, EB, CB), lambda i: (0, i, 0)),
        out_shape=jax.ShapeDtypeStruct((nblk, EP, CB), jnp.float32),
    )


@functools.cache
def _k_apply(cout, relu, has_ident, emit_raw, emit_hp):
    nblk = cout // CB

    def body(*refs):
        it = iter(refs)
        y_ref = next(it)
        ps_ref = next(it)
        pq_ref = next(it)
        g_ref = next(it)
        b_ref = next(it)
        dv_ref = next(it)
        ident_ref = next(it) if has_ident else None
        raw_ref = next(it) if emit_raw else None
        hp_ref = next(it) if emit_hp else None

        m = ps_ref[...].reshape(cout) * (1.0 / N)
        v = pq_ref[...].reshape(cout) * (1.0 / N)
        a = g_ref[...][0] * lax.rsqrt(v + BN_EPS)
        yv = (y_ref[...] - m[None, :]) * a[None, :] + b_ref[...][0][None, :]
        if has_ident:
            yv = yv + ident_ref[...]
        if relu:
            yv = jnp.maximum(yv, 0.0)
        if emit_raw:
            raw_ref[...] = yv
        if emit_hp:
            hv = yv * dv_ref[...][:, 0:1]
            for k in range(nblk):
                hp_ref[k] = hv[:, k * CB:(k + 1) * CB]

    in_specs = [pl.BlockSpec((BN, cout), lambda i: (i, 0)),
                pl.BlockSpec((1, 1, cout), lambda i: (0, 0, 0)),
                pl.BlockSpec((1, 1, cout), lambda i: (0, 0, 0)),
                pl.BlockSpec((1, cout), lambda i: (0, 0)),
                pl.BlockSpec((1, cout), lambda i: (0, 0)),
                pl.BlockSpec((BN, CB), lambda i: (i, 0))]
    if has_ident:
        in_specs.append(pl.BlockSpec((BN, cout), lambda i: (i, 0)))
    out_specs, out_shape = [], []
    if emit_raw:
        out_specs.append(pl.BlockSpec((BN, cout), lambda i: (i, 0)))
        out_shape.append(jax.ShapeDtypeStruct((N_PAD, cout), jnp.float32))
    if emit_hp:
        out_specs.append(pl.BlockSpec((nblk, BN, CB), lambda i: (0, i, 0)))
        out_shape.append(
            jax.ShapeDtypeStruct((nblk, N_PAD, CB), jnp.float32))
    return pl.pallas_call(
        body,
        grid=(NB,),
        in_specs=in_specs,
        out_specs=out_specs,
        out_shape=out_shape,
    )


def _gcn(h_raw, W, b, norm32, srcm, dstm):
    cin, cout = W.shape
    nblk = cout // CB
    hp = _k_premm(cin, cout)(h_raw, W, b.reshape(1, cout))
    msg = _sc_gather(nblk)(hp, srcm)
    msg = _k_msgmul(nblk)(msg, norm32)
    agg = _sc_scatter(nblk)(msg, dstm)
    y, ps = _k_stats(cout)(agg)
    pq = _k_stats2(cout)(y, ps)
    return y, ps, pq


def kernel(x, edge_index, params):
    cin0 = x.shape[1]

    xp = jnp.zeros((N_PAD, cin0), jnp.float32).at[:N].set(x)
    loop = jnp.arange(N, dtype=jnp.int32)
    src = jnp.concatenate([edge_index[0], loop])
    dst = jnp.concatenate([edge_index[1], loop])
    perm = jnp.argsort(dst, stable=True)
    src = src[perm]
    dst = dst[perm]
    padv = jnp.full((EP - E_RAW,), N, jnp.int32)
    srcm = jnp.concatenate([src, padv]).reshape(ROWS, CHUNK)
    dstm = jnp.concatenate([dst, padv]).reshape(ROWS, CHUNK)

    degp = _sc_deg()(dstm)
    dinv32 = _k_dinv()(degp)
    norm32 = _sc_norm()(dinv32, srcm, dstm)

    p = params["conv1"]
    cout = p["W"].shape[1]
    y, ps, pq = _gcn(xp, p["W"], p["b"], norm32, srcm, dstm)
    bn = params["bn1"]
    (h_raw,) = _k_apply(cout, True, False, True, False)(
        y, ps, pq, bn["g"].reshape(1, cout), bn["b"].reshape(1, cout),
        dinv32)

    for blk in params["blocks"]:
        if "down_conv" in blk:
            p = blk["down_conv"]
            cout = p["W"].shape[1]
            y, ps, pq = _gcn(h_raw, p["W"], p["b"], norm32, srcm, dstm)
            bn = blk["down_bn"]
            (h_raw,) = _k_apply(cout, False, False, True, False)(
                y, ps, pq, bn["g"].reshape(1, cout),
                bn["b"].reshape(1, cout), dinv32)
        ident = h_raw
        p = blk["conv1"]
        cout = p["W"].shape[1]
        y, ps, pq = _gcn(h_raw, p["W"], p["b"], norm32, srcm, dstm)
        bn = blk["bn1"]
        (h_t,) = _k_apply(cout, True, False, True, False)(
            y, ps, pq, bn["g"].reshape(1, cout), bn["b"].reshape(1, cout),
            dinv32)
        p = blk["conv2"]
        y, ps, pq = _gcn(h_t, p["W"], p["b"], norm32, srcm, dstm)
        bn = blk["bn2"]
        (h_raw,) = _k_apply(cout, True, True, True, False)(
            y, ps, pq, bn["g"].reshape(1, cout), bn["b"].reshape(1, cout),
            dinv32, ident)

    return h_raw[:N]


# speedup vs baseline: 2.6362x; 2.5001x over previous
import functools

import jax
import jax.numpy as jnp
from jax import lax
from jax.experimental import pallas as pl
from jax.experimental.pallas import tpu as pltpu
from jax.experimental.pallas import tpu_sc as plsc

N = 50000
N_PAD = 50176
BN = 512
NB = N_PAD // BN
CB = 32
CHUNK = 128
SUPER = 16
E_RAW = 850000
EP = ((E_RAW + 16 * CHUNK - 1) // (16 * CHUNK)) * (16 * CHUNK)
ROWS = EP // CHUNK
TILE_ROWS = ROWS // 16
WID_ROWS = ROWS // 32
NSUP_TILE = TILE_ROWS // SUPER
NSUP_WID = WID_ROWS // SUPER
PT = N_PAD // 16
NZ = 32
ZR = PT // NZ
NBUF = 4
BN_EPS = 1e-5


def _sc_deg_kernel():
    mesh = plsc.VectorSubcoreMesh(core_axis_name="c", subcore_axis_name="s")
    scratch = [
        pltpu.VMEM_SHARED((N_PAD, CB), jnp.float32),
        pltpu.VMEM((ZR, CB), jnp.float32),
        pltpu.VMEM((SUPER, CHUNK), jnp.int32),
        pltpu.VMEM((1, CHUNK, CB), jnp.float32),
    ]

    def body(dstm, out, acc, zbuf, didx, obuf):
        c = lax.axis_index("c")
        s = lax.axis_index("s")

        def zrow(i, _):
            zv = jnp.zeros((16,), jnp.float32)
            zbuf[i, pl.ds(0, 16)] = zv
            zbuf[i, pl.ds(16, 16)] = zv
            return 0
        lax.fori_loop(0, ZR, zrow, 0)

        def orow(i, _):
            ov = jnp.full((16,), 1.0, jnp.float32)
            obuf[0, i, pl.ds(0, 16)] = ov
            obuf[0, i, pl.ds(16, 16)] = ov
            return 0
        lax.fori_loop(0, CHUNK, orow, 0)

        for z in range(NZ):
            pltpu.sync_copy(zbuf, acc.at[pl.ds(s * PT + z * ZR, ZR), :])
        plsc.subcore_barrier()

        rows0 = (s * 2 + c) * WID_ROWS

        def super_body(u, _):
            r0 = rows0 + u * SUPER
            pltpu.sync_copy(dstm.at[pl.ds(r0, SUPER), :], didx)
            for j in range(SUPER):
                pltpu.sync_copy(obuf.at[0], acc.at[didx.at[j]], add=True)
            return 0
        lax.fori_loop(0, NSUP_WID, super_body, 0)
        plsc.subcore_barrier()

        for z in range(NZ):
            r = s * PT + z * ZR
            pltpu.sync_copy(acc.at[pl.ds(r, ZR), :],
                            out.at[c, pl.ds(r, ZR), :])
        plsc.subcore_barrier()

    return pl.kernel(
        body, out_type=jax.ShapeDtypeStruct((2, N_PAD, CB), jnp.float32),
        mesh=mesh, scratch_types=scratch,
        compiler_params=pltpu.CompilerParams(use_tc_tiling_on_sc=False))


def _sc_norm_kernel():
    mesh = plsc.VectorSubcoreMesh(core_axis_name="c", subcore_axis_name="s")
    scratch = [
        pltpu.VMEM((1, CHUNK), jnp.int32),
        pltpu.VMEM((1, CHUNK), jnp.int32),
        pltpu.VMEM((CHUNK, CB), jnp.float32),
        pltpu.VMEM((CHUNK, CB), jnp.float32),
        pltpu.SemaphoreType.DMA,
        pltpu.SemaphoreType.DMA,
    ]

    def body(dinv_t, srcm, dstm, out, sidx, didx, gs, gd, sem1, sem2):
        c = lax.axis_index("c")
        s = lax.axis_index("s")
        rows0 = (s * 2 + c) * WID_ROWS

        def chunk_body(u, _):
            r = rows0 + u
            pltpu.sync_copy(srcm.at[pl.ds(r, 1), :], sidx)
            pltpu.sync_copy(dstm.at[pl.ds(r, 1), :], didx)
            h1 = pltpu.async_copy(dinv_t.at[sidx.at[0]], gs, sem1)
            h2 = pltpu.async_copy(dinv_t.at[didx.at[0]], gd, sem2)
            h1.wait()
            h2.wait()

            def mrow(i, _):
                a0 = gs[i, pl.ds(0, 16)]
                b0 = gd[i, pl.ds(0, 16)]
                gs[i, pl.ds(0, 16)] = a0 * b0
                a1 = gs[i, pl.ds(16, 16)]
                b1 = gd[i, pl.ds(16, 16)]
                gs[i, pl.ds(16, 16)] = a1 * b1
                return 0
            lax.fori_loop(0, CHUNK, mrow, 0)
            pltpu.sync_copy(gs, out.at[pl.ds(r * CHUNK, CHUNK), :])
            return 0
        lax.fori_loop(0, WID_ROWS, chunk_body, 0)

    return pl.kernel(
        body, out_type=jax.ShapeDtypeStruct((EP, CB), jnp.float32),
        mesh=mesh, scratch_types=scratch,
        compiler_params=pltpu.CompilerParams(use_tc_tiling_on_sc=False))


def _sc_gather_kernel(nblk):
    mesh = plsc.VectorSubcoreMesh(core_axis_name="c", subcore_axis_name="s")
    scratch = [
        pltpu.VMEM((SUPER, CHUNK), jnp.int32),
        pltpu.VMEM((NBUF, CHUNK, CB), jnp.float32),
    ] + [pltpu.SemaphoreType.DMA] * NBUF

    def body(table, srcm, out, sidx, gbuf, *sems):
        c = lax.axis_index("c")
        s = lax.axis_index("s")
        rows0 = s * TILE_ROWS

        for bi in range(nblk // 2):
            blk = bi * 2 + c

            def super_body(u, _):
                r0 = rows0 + u * SUPER
                pltpu.sync_copy(srcm.at[pl.ds(r0, SUPER), :], sidx)
                handles = {}
                for j in range(NBUF):
                    handles[j] = pltpu.async_copy(
                        table.at[blk].at[sidx.at[j]], gbuf.at[j], sems[j])
                for j in range(SUPER):
                    handles[j].wait()
                    pltpu.sync_copy(
                        gbuf.at[j % NBUF],
                        out.at[blk, pl.ds((r0 + j) * CHUNK, CHUNK), :])
                    nx = j + NBUF
                    if nx < SUPER:
                        handles[nx] = pltpu.async_copy(
                            table.at[blk].at[sidx.at[nx]],
                            gbuf.at[nx % NBUF], sems[nx % NBUF])
                return 0
            lax.fori_loop(0, NSUP_TILE, super_body, 0)

    return pl.kernel(
        body, out_type=jax.ShapeDtypeStruct((nblk, EP, CB), jnp.float32),
        mesh=mesh, scratch_types=scratch,
        compiler_params=pltpu.CompilerParams(use_tc_tiling_on_sc=False))


def _sc_scatter_kernel(nblk):
    mesh = plsc.VectorSubcoreMesh(core_axis_name="c", subcore_axis_name="s")
    scratch = [
        pltpu.VMEM_SHARED((N_PAD, CB), jnp.float32),
        pltpu.VMEM((ZR, CB), jnp.float32),
        pltpu.VMEM((SUPER, CHUNK), jnp.int32),
        pltpu.VMEM((NBUF, CHUNK, CB), jnp.float32),
    ] + [pltpu.SemaphoreType.DMA] * NBUF

    def body(msg, dstm, out, acc, zbuf, didx, lbuf, *sems):
        c = lax.axis_index("c")
        s = lax.axis_index("s")

        def zrow(i, _):
            zv = jnp.zeros((16,), jnp.float32)
            zbuf[i, pl.ds(0, 16)] = zv
            zbuf[i, pl.ds(16, 16)] = zv
            return 0
        lax.fori_loop(0, ZR, zrow, 0)

        rows0 = s * TILE_ROWS

        for bi in range(nblk // 2):
            blk = bi * 2 + c

            for z in range(NZ):
                pltpu.sync_copy(zbuf, acc.at[pl.ds(s * PT + z * ZR, ZR), :])
            plsc.subcore_barrier()

            def super_body(u, _):
                r0 = rows0 + u * SUPER
                pltpu.sync_copy(dstm.at[pl.ds(r0, SUPER), :], didx)
                handles = {}
                for j in range(NBUF):
                    handles[j] = pltpu.async_copy(
                        msg.at[blk, pl.ds((r0 + j) * CHUNK, CHUNK), :],
                        lbuf.at[j], sems[j])
                for j in range(SUPER):
                    handles[j].wait()
                    pltpu.sync_copy(lbuf.at[j % NBUF],
                                    acc.at[didx.at[j]], add=True)
                    nx = j + NBUF
                    if nx < SUPER:
                        handles[nx] = pltpu.async_copy(
                            msg.at[blk, pl.ds((r0 + nx) * CHUNK, CHUNK), :],
                            lbuf.at[nx % NBUF], sems[nx % NBUF])
                return 0
            lax.fori_loop(0, NSUP_TILE, super_body, 0)
            plsc.subcore_barrier()

            for z in range(NZ):
                r = s * PT + z * ZR
                pltpu.sync_copy(acc.at[pl.ds(r, ZR), :],
                                out.at[blk, pl.ds(r, ZR), :])
            plsc.subcore_barrier()

    return pl.kernel(
        body, out_type=jax.ShapeDtypeStruct((nblk, N_PAD, CB), jnp.float32),
        mesh=mesh, scratch_types=scratch,
        compiler_params=pltpu.CompilerParams(use_tc_tiling_on_sc=False))


def _sc_gcnagg_kernel(nblk):
    mesh = plsc.VectorSubcoreMesh(core_axis_name="c", subcore_axis_name="s")
    scratch = [
        pltpu.VMEM_SHARED((N_PAD, CB), jnp.float32),
        pltpu.VMEM((ZR, CB), jnp.float32),
        pltpu.VMEM((SUPER, CHUNK), jnp.int32),
        pltpu.VMEM((SUPER, CHUNK), jnp.int32),
        pltpu.VMEM((NBUF, CHUNK, CB), jnp.float32),
        pltpu.VMEM((CHUNK, CB), jnp.float32),
    ] + [pltpu.SemaphoreType.DMA] * NBUF

    def body(table, norm, srcm, dstm, out, acc, zbuf, sidx, didx,
             gbuf, nbuf, *sems):
        c = lax.axis_index("c")
        s = lax.axis_index("s")

        def zrow(i, _):
            zv = jnp.zeros((16,), jnp.float32)
            zbuf[i, pl.ds(0, 16)] = zv
            zbuf[i, pl.ds(16, 16)] = zv
            return 0
        lax.fori_loop(0, ZR, zrow, 0)

        rows0 = s * TILE_ROWS

        for bi in range(nblk // 2):
            blk = bi * 2 + c

            for z in range(NZ):
                pltpu.sync_copy(zbuf, acc.at[pl.ds(s * PT + z * ZR, ZR), :])
            plsc.subcore_barrier()

            def super_body(u, _):
                r0 = rows0 + u * SUPER
                pltpu.sync_copy(srcm.at[pl.ds(r0, SUPER), :], sidx)
                pltpu.sync_copy(dstm.at[pl.ds(r0, SUPER), :], didx)
                handles = {}
                for j in range(NBUF):
                    handles[j] = pltpu.async_copy(
                        table.at[blk].at[sidx.at[j]], gbuf.at[j], sems[j])
                for j in range(SUPER):
                    handles[j].wait()
                    pltpu.sync_copy(
                        norm.at[pl.ds((r0 + j) * CHUNK, CHUNK), :], nbuf)
                    g = gbuf.at[j % NBUF]

                    def mrow(i, _):
                        g[i, pl.ds(0, 16)] = (g[i, pl.ds(0, 16)]
                                              * nbuf[i, pl.ds(0, 16)])
                        g[i, pl.ds(16, 16)] = (g[i, pl.ds(16, 16)]
                                               * nbuf[i, pl.ds(16, 16)])
                        return 0
                    lax.fori_loop(0, CHUNK, mrow, 0)
                    pltpu.sync_copy(g, acc.at[didx.at[j]], add=True)
                    nx = j + NBUF
                    if nx < SUPER:
                        handles[nx] = pltpu.async_copy(
                            table.at[blk].at[sidx.at[nx]],
                            gbuf.at[nx % NBUF], sems[nx % NBUF])
                return 0
            lax.fori_loop(0, NSUP_TILE, super_body, 0)
            plsc.subcore_barrier()

            for z in range(NZ):
                r = s * PT + z * ZR
                pltpu.sync_copy(acc.at[pl.ds(r, ZR), :],
                                out.at[blk, pl.ds(r, ZR), :])
            plsc.subcore_barrier()

    return pl.kernel(
        body, out_type=jax.ShapeDtypeStruct((nblk, N_PAD, CB), jnp.float32),
        mesh=mesh, scratch_types=scratch,
        compiler_params=pltpu.CompilerParams(use_tc_tiling_on_sc=False))


@functools.cache
def _sc_gcnagg(nblk):
    return _sc_gcnagg_kernel(nblk)


@functools.cache
def _sc_deg():
    return _sc_deg_kernel()


@functools.cache
def _sc_norm():
    return _sc_norm_kernel()


@functools.cache
def _sc_gather(nblk):
    return _sc_gather_kernel(nblk)


@functools.cache
def _sc_scatter(nblk):
    return _sc_scatter_kernel(nblk)


def _iota_rows():
    return lax.broadcasted_iota(jnp.int32, (BN, CB), 0)


@functools.cache
def _k_dinv():
    def body(deg_ref, dv_ref):
        i = pl.program_id(0)
        d = deg_ref[0] + deg_ref[1]
        dv = lax.rsqrt(jnp.maximum(d, 1.0))
        row = i * BN + _iota_rows()
        dv_ref[...] = jnp.where(row < N, dv, 0.0)
    return pl.pallas_call(
        body,
        grid=(NB,),
        in_specs=[pl.BlockSpec((2, BN, CB), lambda i: (0, i, 0))],
        out_specs=pl.BlockSpec((BN, CB), lambda i: (i, 0)),
        out_shape=jax.ShapeDtypeStruct((N_PAD, CB), jnp.float32),
    )


@functools.cache
def _k_premm(cin, cout):
    nblk = cout // CB

    def body(x_ref, w_ref, b_ref, hp_ref):
        y = jnp.dot(x_ref[...], w_ref[...],
                    preferred_element_type=jnp.float32)
        hv = y + b_ref[...]
        for k in range(nblk):
            hp_ref[k] = hv[:, k * CB:(k + 1) * CB]
    return pl.pallas_call(
        body,
        grid=(NB,),
        in_specs=[pl.BlockSpec((BN, cin), lambda i: (i, 0)),
                  pl.BlockSpec((cin, cout), lambda i: (0, 0)),
                  pl.BlockSpec((1, cout), lambda i: (0, 0))],
        out_specs=pl.BlockSpec((nblk, BN, CB), lambda i: (0, i, 0)),
        out_shape=jax.ShapeDtypeStruct((nblk, N_PAD, CB), jnp.float32),
    )


@functools.cache
def _k_stats(cout):
    nblk_in = cout // CB

    def body(agg_ref, y_ref, ps_ref, acc):
        i = pl.program_id(0)
        y = jnp.concatenate([agg_ref[k] for k in range(nblk_in)], axis=1)
        y_ref[...] = y

        @pl.when(i == 0)
        def _():
            acc[...] = jnp.zeros((4, 128, cout), jnp.float32)
        for k in range(4):
            acc[k] = acc[k] + y[k * 128:(k + 1) * 128, :]

        @pl.when(i == NB - 1)
        def _():
            a = acc[0]
            for k in range(1, 4):
                a = a + acc[k]
            for st in [1, 2, 4, 8, 16, 32, 64]:
                a = a + pltpu.roll(a, 128 - st, 0)
            ps_ref[...] = a[0].reshape(1, 1, cout)

    return pl.pallas_call(
        body,
        grid=(NB,),
        in_specs=[pl.BlockSpec((nblk_in, BN, CB), lambda i: (0, i, 0))],
        out_specs=[pl.BlockSpec((BN, cout), lambda i: (i, 0)),
                   pl.BlockSpec((1, 1, cout), lambda i: (0, 0, 0))],
        out_shape=[jax.ShapeDtypeStruct((N_PAD, cout), jnp.float32),
                   jax.ShapeDtypeStruct((1, 1, cout), jnp.float32)],
        scratch_shapes=[pltpu.VMEM((4, 128, cout), jnp.float32)],
    )


@functools.cache
def _k_stats2(cout):
    def body(y_ref, ps_ref, pq_ref, acc):
        i = pl.program_id(0)
        m = ps_ref[...].reshape(cout) * (1.0 / N)
        c = y_ref[...] - m[None, :]
        row = i * BN + lax.broadcasted_iota(jnp.int32, (BN, cout), 0)
        sq = jnp.where(row < N, c * c, 0.0)

        @pl.when(i == 0)
        def _():
            acc[...] = jnp.zeros((4, 128, cout), jnp.float32)
        for k in range(4):
            acc[k] = acc[k] + sq[k * 128:(k + 1) * 128, :]

        @pl.when(i == NB - 1)
        def _():
            a = acc[0]
            for k in range(1, 4):
                a = a + acc[k]
            for st in [1, 2, 4, 8, 16, 32, 64]:
                a = a + pltpu.roll(a, 128 - st, 0)
            pq_ref[...] = a[0].reshape(1, 1, cout)

    return pl.pallas_call(
        body,
        grid=(NB,),
        in_specs=[pl.BlockSpec((BN, cout), lambda i: (i, 0)),
                  pl.BlockSpec((1, 1, cout), lambda i: (0, 0, 0))],
        out_specs=pl.BlockSpec((1, 1, cout), lambda i: (0, 0, 0)),
        out_shape=jax.ShapeDtypeStruct((1, 1, cout), jnp.float32),
        scratch_shapes=[pltpu.VMEM((4, 128, cout), jnp.float32)],
    )


@functools.cache
def _k_msgmul(nblk):
    EB = 512

    def body(g_ref, n_ref, o_ref):
        nv = n_ref[...][:, 0:1]
        for k in range(nblk):
            o_ref[k] = g_ref[k] * nv

    return pl.pallas_call(
        body,
        grid=(EP // EB,),
        in_specs=[pl.BlockSpec((nblk, EB, CB), lambda i: (0, i, 0)),
                  pl.BlockSpec((EB, CB), lambda i: (i, 0))],
        out_specs=pl.BlockSpec((nblk, EB, CB), lambda i: (0, i, 0)),
        out_shape=jax.ShapeDtypeStruct((nblk, EP, CB), jnp.float32),
    )


@functools.cache
def _k_apply(cout, relu, has_ident, emit_raw, emit_hp):
    nblk = cout // CB

    def body(*refs):
        it = iter(refs)
        y_ref = next(it)
        ps_ref = next(it)
        pq_ref = next(it)
        g_ref = next(it)
        b_ref = next(it)
        dv_ref = next(it)
        ident_ref = next(it) if has_ident else None
        raw_ref = next(it) if emit_raw else None
        hp_ref = next(it) if emit_hp else None

        m = ps_ref[...].reshape(cout) * (1.0 / N)
        v = pq_ref[...].reshape(cout) * (1.0 / N)
        a = g_ref[...][0] * lax.rsqrt(v + BN_EPS)
        yv = (y_ref[...] - m[None, :]) * a[None, :] + b_ref[...][0][None, :]
        if has_ident:
            yv = yv + ident_ref[...]
        if relu:
            yv = jnp.maximum(yv, 0.0)
        if emit_raw:
            raw_ref[...] = yv
        if emit_hp:
            hv = yv * dv_ref[...][:, 0:1]
            for k in range(nblk):
                hp_ref[k] = hv[:, k * CB:(k + 1) * CB]

    in_specs = [pl.BlockSpec((BN, cout), lambda i: (i, 0)),
                pl.BlockSpec((1, 1, cout), lambda i: (0, 0, 0)),
                pl.BlockSpec((1, 1, cout), lambda i: (0, 0, 0)),
                pl.BlockSpec((1, cout), lambda i: (0, 0)),
                pl.BlockSpec((1, cout), lambda i: (0, 0)),
                pl.BlockSpec((BN, CB), lambda i: (i, 0))]
    if has_ident:
        in_specs.append(pl.BlockSpec((BN, cout), lambda i: (i, 0)))
    out_specs, out_shape = [], []
    if emit_raw:
        out_specs.append(pl.BlockSpec((BN, cout), lambda i: (i, 0)))
        out_shape.append(jax.ShapeDtypeStruct((N_PAD, cout), jnp.float32))
    if emit_hp:
        out_specs.append(pl.BlockSpec((nblk, BN, CB), lambda i: (0, i, 0)))
        out_shape.append(
            jax.ShapeDtypeStruct((nblk, N_PAD, CB), jnp.float32))
    return pl.pallas_call(
        body,
        grid=(NB,),
        in_specs=in_specs,
        out_specs=out_specs,
        out_shape=out_shape,
    )


def _gcn(h_raw, W, b, norm32, srcm, dstm):
    cin, cout = W.shape
    nblk = cout // CB
    hp = _k_premm(cin, cout)(h_raw, W, b.reshape(1, cout))
    agg = _sc_gcnagg(nblk)(hp, norm32, srcm, dstm)
    y, ps = _k_stats(cout)(agg)
    pq = _k_stats2(cout)(y, ps)
    return y, ps, pq


def kernel(x, edge_index, params):
    cin0 = x.shape[1]

    xp = jnp.zeros((N_PAD, cin0), jnp.float32).at[:N].set(x)
    loop = jnp.arange(N, dtype=jnp.int32)
    src = jnp.concatenate([edge_index[0], loop])
    dst = jnp.concatenate([edge_index[1], loop])
    perm = jnp.argsort(dst, stable=True)
    src = src[perm]
    dst = dst[perm]
    padv = jnp.full((EP - E_RAW,), N, jnp.int32)
    srcm = jnp.concatenate([src, padv]).reshape(ROWS, CHUNK)
    dstm = jnp.concatenate([dst, padv]).reshape(ROWS, CHUNK)

    degp = _sc_deg()(dstm)
    dinv32 = _k_dinv()(degp)
    norm32 = _sc_norm()(dinv32, srcm, dstm)

    p = params["conv1"]
    cout = p["W"].shape[1]
    y, ps, pq = _gcn(xp, p["W"], p["b"], norm32, srcm, dstm)
    bn = params["bn1"]
    (h_raw,) = _k_apply(cout, True, False, True, False)(
        y, ps, pq, bn["g"].reshape(1, cout), bn["b"].reshape(1, cout),
        dinv32)

    for blk in params["blocks"]:
        if "down_conv" in blk:
            p = blk["down_conv"]
            cout = p["W"].shape[1]
            y, ps, pq = _gcn(h_raw, p["W"], p["b"], norm32, srcm, dstm)
            bn = blk["down_bn"]
            (h_raw,) = _k_apply(cout, False, False, True, False)(
                y, ps, pq, bn["g"].reshape(1, cout),
                bn["b"].reshape(1, cout), dinv32)
        ident = h_raw
        p = blk["conv1"]
        cout = p["W"].shape[1]
        y, ps, pq = _gcn(h_raw, p["W"], p["b"], norm32, srcm, dstm)
        bn = blk["bn1"]
        (h_t,) = _k_apply(cout, True, False, True, False)(
            y, ps, pq, bn["g"].reshape(1, cout), bn["b"].reshape(1, cout),
            dinv32)
        p = blk["conv2"]
        y, ps, pq = _gcn(h_t, p["W"], p["b"], norm32, srcm, dstm)
        bn = blk["bn2"]
        (h_raw,) = _k_apply(cout, True, True, True, False)(
            y, ps, pq, bn["g"].reshape(1, cout), bn["b"].reshape(1, cout),
            dinv32, ident)

    return h_raw[:N]
